# Initial kernel scaffold; baseline (speedup 1.0000x reference)
import jax
import jax.numpy as jnp
from jax.experimental import pallas as pl


def kernel(vec_binary, vec_cfg, vec_fcg, W1, b1, W2, b2, gate_w, gate_b, fus_w, fus_b, cls_w, cls_b):
    raise NotImplementedError("write your pallas kernel here")


# speedup vs baseline: 1.9774x; 1.9774x over previous
import jax
import jax.numpy as jnp
from jax import lax
from jax.experimental import pallas as pl
from jax.experimental.pallas import tpu as pltpu

B, D, H, O, E, FUS, C = 2048, 1024, 512, 512, 8, 1024, 2
BB = 256
NBLK = B // BB


def _softmax(l):
    m = jnp.max(l, axis=1, keepdims=True)
    e = jnp.exp(l - m)
    return e / jnp.sum(e, axis=1, keepdims=True)


def _top2_weights(g):
    iota = lax.broadcasted_iota(jnp.int32, g.shape, 1)
    m1 = jnp.max(g, axis=1, keepdims=True)
    i1 = jnp.min(jnp.where(g == m1, iota, E), axis=1, keepdims=True)
    mask1 = iota == i1
    gm = jnp.where(mask1, -jnp.inf, g)
    m2 = jnp.max(gm, axis=1, keepdims=True)
    i2 = jnp.min(jnp.where(gm == m2, iota, E), axis=1, keepdims=True)
    mask2 = iota == i2
    s = m1 + m2
    cw = jnp.where(mask1, m1 / s, 0.0) + jnp.where(mask2, m2 / s, 0.0)
    sm = (mask1 | mask2).astype(g.dtype)
    return cw, sm


def _fused_kernel(xb_ref, xc_ref, W1_ref, b1_ref, W2_ref, b2_ref, gw_ref,
                  gb_ref, fw_ref, fb_ref, cw_ref, cb_ref,
                  out_ref, dist_ref, eq_ref,
                  esum_ref, cntc_ref, cntr_ref, rhr_ref):
    i = pl.program_id(0)

    @pl.when(i == 0)
    def _init():
        esum_ref[...] = jnp.zeros_like(esum_ref)
        cntc_ref[...] = jnp.zeros_like(cntc_ref)
        cntr_ref[...] = jnp.zeros_like(cntr_ref)
        rhr_ref[...] = jnp.zeros_like(rhr_ref)

    gw = gw_ref[...]
    gb = gb_ref[...]
    ones_col = jnp.ones((BB, 1), jnp.float32)

    finals = []
    for x_ref, factor in ((xb_ref, 1.0), (xc_ref, 2.0)):
        x = x_ref[...]
        g = _softmax(jnp.dot(x, gw, preferred_element_type=jnp.float32) + gb)
        cw, sm = _top2_weights(g)
        fin = jnp.zeros((BB, O), jnp.float32)
        rows = []
        for e in range(E):
            h = jnp.maximum(
                jnp.dot(x, W1_ref[e], preferred_element_type=jnp.float32)
                + b1_ref[e:e + 1, :], 0.0)
            eo = (jnp.dot(h, W2_ref[e], preferred_element_type=jnp.float32)
                  + b2_ref[e:e + 1, :])
            fin = fin + cw[:, e:e + 1] * eo
            rows.append(lax.dot_general(sm[:, e:e + 1], eo,
                                        (((0,), (0,)), ((), ())),
                                        preferred_element_type=jnp.float32))
        esum_ref[...] += factor * jnp.concatenate(rows, axis=0)
        cntr_ref[...] += factor * jnp.sum(sm, axis=0, keepdims=True)
        cntc_ref[...] += factor * lax.dot_general(
            sm, ones_col, (((0,), (0,)), ((), ())),
            preferred_element_type=jnp.float32)
        rhr_ref[...] += factor * jnp.sum(g, axis=0, keepdims=True)
        finals.append(fin)

    fin_b, fin_c = finals
    f0 = fw_ref[0:O, :]
    f12 = fw_ref[O:2 * O, :] + fw_ref[2 * O:3 * O, :]
    fused = jnp.maximum(
        jnp.dot(fin_b, f0, preferred_element_type=jnp.float32)
        + jnp.dot(fin_c, f12, preferred_element_type=jnp.float32)
        + fb_ref[...], 0.0)
    out_ref[...] = (jnp.dot(fused, cw_ref[...],
                            preferred_element_type=jnp.float32) + cb_ref[...])

    @pl.when(i == NBLK - 1)
    def _losses():
        cnt_c = cntc_ref[...]
        cnt_r = cntr_ref[...]
        eq = jnp.sum(cnt_r * rhr_ref[...]) * (1.0 / E)
        avg = esum_ref[...] / jnp.maximum(cnt_c, 1.0)
        G = lax.dot_general(avg, avg, (((1,), (1,)), ((), ())),
                            preferred_element_type=jnp.float32)
        r0 = lax.broadcasted_iota(jnp.int32, (E, E), 0)
        r1 = lax.broadcasted_iota(jnp.int32, (E, E), 1)
        eye = r0 == r1
        Gd = jnp.where(eye, G, 0.0)
        diag_c = jnp.sum(Gd, axis=1, keepdims=True)
        diag_r = jnp.sum(Gd, axis=0, keepdims=True)
        d2 = diag_c + diag_r - 2.0 * G
        sim = jnp.exp(-0.5 * d2)
        pm = (~eye) & (cnt_c > 0.0) & (cnt_r > 0.0)
        npairs = jnp.sum(jnp.where(pm, 1.0, 0.0)) * 0.5
        ssum = jnp.sum(jnp.where(pm, sim, 0.0)) * 0.5
        dist_ref[0, 0] = -ssum / jnp.maximum(npairs, 1.0)
        eq_ref[0, 0] = eq


def kernel(vec_binary, vec_cfg, vec_fcg, W1, b1, W2, b2, gate_w, gate_b,
           fus_w, fus_b, cls_w, cls_b):
    del vec_fcg
    full = lambda *shape: pl.BlockSpec(shape, lambda i: (0,) * len(shape))
    out, dist, eq = pl.pallas_call(
        _fused_kernel,
        grid=(NBLK,),
        in_specs=[
            pl.BlockSpec((BB, D), lambda i: (i, 0)),
            pl.BlockSpec((BB, D), lambda i: (i, 0)),
            full(E, D, H),
            full(E, H),
            full(E, H, O),
            full(E, O),
            full(D, E),
            full(1, E),
            full(3 * O, FUS),
            full(1, FUS),
            full(FUS, C),
            full(1, C),
        ],
        out_specs=[
            pl.BlockSpec((BB, C), lambda i: (i, 0)),
            pl.BlockSpec((1, 1), lambda i: (0, 0), memory_space=pltpu.SMEM),
            pl.BlockSpec((1, 1), lambda i: (0, 0), memory_space=pltpu.SMEM),
        ],
        out_shape=[
            jax.ShapeDtypeStruct((B, C), jnp.float32),
            jax.ShapeDtypeStruct((1, 1), jnp.float32),
            jax.ShapeDtypeStruct((1, 1), jnp.float32),
        ],
        scratch_shapes=[
            pltpu.VMEM((E, O), jnp.float32),
            pltpu.VMEM((E, 1), jnp.float32),
            pltpu.VMEM((1, E), jnp.float32),
            pltpu.VMEM((1, E), jnp.float32),
        ],
        compiler_params=pltpu.CompilerParams(
            dimension_semantics=("arbitrary",)),
    )(vec_binary, vec_cfg, W1, b1, W2, b2, gate_w,
      gate_b.reshape(1, E), fus_w, fus_b.reshape(1, FUS), cls_w,
      cls_b.reshape(1, C))
    return out, dist.reshape(()), eq.reshape(())
